# baseline (speedup 1.0000x reference)
import functools

import jax
import jax.numpy as jnp
from jax import lax
from jax.experimental import pallas as pl
from jax.experimental.pallas import tpu as pltpu
from jax.experimental.pallas import tpu_sc as plsc

D_MODEL = 64
SCALE = 8.0
NC, NS = 2, 16
NW = NC * NS
CH = 128
L = 16


@functools.cache
def _make_kernel(B):
    assert B % (NW * CH) == 0
    nch = B // (NW * CH)
    per_w = nch * CH
    mesh = plsc.VectorSubcoreMesh(core_axis_name="c", subcore_axis_name="s")

    @functools.partial(
        pl.kernel,
        out_type=jax.ShapeDtypeStruct((B, D_MODEL), jnp.float32),
        mesh=mesh,
        scratch_types=[
            pltpu.VMEM((nch, CH), jnp.int32),
            pltpu.VMEM((CH, D_MODEL), jnp.float32),
            pltpu.SemaphoreType.DMA,
        ],
        compiler_params=pltpu.CompilerParams(use_tc_tiling_on_sc=False),
    )
    def emb_kernel(x_hbm, lut_hbm, out_hbm, idx_v, rows_v, sem):
        wid = lax.axis_index("s") * NC + lax.axis_index("c")
        pltpu.sync_copy(x_hbm.at[pl.ds(wid * nch, nch)], idx_v)

        def chunk_body(j, carry):
            pltpu.async_copy(lut_hbm.at[idx_v.at[j]], rows_v, sem).wait()

            def row_body(r, c2):
                for c in range(D_MODEL // L):
                    rows_v[r, pl.ds(c * L, L)] = rows_v[r, pl.ds(c * L, L)] * SCALE
                return c2

            lax.fori_loop(0, CH, row_body, 0)
            pltpu.sync_copy(rows_v, out_hbm.at[pl.ds(wid * per_w + j * CH, CH)])
            return carry

        lax.fori_loop(0, nch, chunk_body, 0)

    return emb_kernel


def kernel(x, lut):
    n, s = x.shape
    B = n * s
    x2d = x.reshape(B // CH, CH).astype(jnp.int32)
    out = _make_kernel(B)(x2d, lut)
    return out.reshape(n, s, D_MODEL)


# speedup vs baseline: 1.2113x; 1.2113x over previous
import functools

import jax
import jax.numpy as jnp
from jax import lax
from jax.experimental import pallas as pl
from jax.experimental.pallas import tpu as pltpu
from jax.experimental.pallas import tpu_sc as plsc

D_MODEL = 64
SCALE = 8.0
NC, NS = 2, 16
NW = NC * NS
CH = 128
L = 16
NBUF = 4


@functools.cache
def _make_kernel(B):
    assert B % (NW * CH) == 0
    nch = B // (NW * CH)
    assert nch % NBUF == 0
    ngrp = nch // NBUF
    per_w = nch * CH
    mesh = plsc.VectorSubcoreMesh(core_axis_name="c", subcore_axis_name="s")

    @functools.partial(
        pl.kernel,
        out_type=jax.ShapeDtypeStruct((B, D_MODEL), jnp.float32),
        mesh=mesh,
        scratch_types=[
            pltpu.VMEM((nch, CH), jnp.int32),
            [pltpu.VMEM((CH, D_MODEL), jnp.float32)] * NBUF,
            [pltpu.VMEM((CH, D_MODEL), jnp.float32)] * NBUF,
            [pltpu.SemaphoreType.DMA] * NBUF,
            [pltpu.SemaphoreType.DMA] * NBUF,
        ],
        compiler_params=pltpu.CompilerParams(use_tc_tiling_on_sc=False),
    )
    def emb_kernel(x_hbm, lut_hbm, out_hbm, idx_v, gbuf, sbuf, gsem, ssem):
        wid = lax.axis_index("s") * NC + lax.axis_index("c")
        pltpu.sync_copy(x_hbm.at[pl.ds(wid * nch, nch)], idx_v)

        def fire_gather(j, b):
            pltpu.async_copy(lut_hbm.at[idx_v.at[j]], gbuf[b], gsem[b])

        def wait_gather(b):
            pltpu.make_async_copy(out_hbm.at[pl.ds(0, CH)], gbuf[b], gsem[b]).wait()

        def fire_scatter(j, b):
            dst = out_hbm.at[pl.ds(wid * per_w + j * CH, CH)]
            pltpu.async_copy(sbuf[b], dst, ssem[b])

        def wait_scatter(b):
            pltpu.make_async_copy(out_hbm.at[pl.ds(0, CH)], sbuf[b], ssem[b]).wait()

        for b in range(NBUF):
            fire_gather(b, b)

        def group_body(g, carry):
            for b in range(NBUF):
                j = g * NBUF + b

                @pl.when(g > 0)
                def _():
                    wait_scatter(b)

                wait_gather(b)

                @plsc.parallel_loop(0, CH, step=1, unroll=4)
                def _(r):
                    for c in range(D_MODEL // L):
                        sl = pl.ds(c * L, L)
                        sbuf[b][r, sl] = gbuf[b][r, sl] * SCALE

                fire_scatter(j, b)

                @pl.when(g + 1 < ngrp)
                def _():
                    fire_gather(j + NBUF, b)

            return carry

        lax.fori_loop(0, ngrp, group_body, 0)
        for b in range(NBUF):
            wait_scatter(b)

    return emb_kernel


def kernel(x, lut):
    n, s = x.shape
    B = n * s
    x2d = x.reshape(B // CH, CH).astype(jnp.int32)
    out = _make_kernel(B)(x2d, lut)
    return out.reshape(n, s, D_MODEL)
